# Initial kernel scaffold; baseline (speedup 1.0000x reference)
import jax
import jax.numpy as jnp
from jax.experimental import pallas as pl


def kernel(fea, ring_n, pool_idx, W, b, gamma, beta):
    raise NotImplementedError("write your pallas kernel here")


# speedup vs baseline: 2283.9975x; 2283.9975x over previous
import functools

import jax
import jax.numpy as jnp
from jax import lax
from jax.experimental import pallas as pl
from jax.experimental.pallas import tpu as pltpu
from jax.experimental.pallas import tpu_sc as plsc

M, C_IN, C_OUT, F, FP, K = 2, 128, 128, 50000, 25000, 16
G = K + 1

NC, NS = 2, 16
NW = NC * NS
NB = 32
FP_PAD = 25088
TOT = M * FP_PAD
RPT = TOT // NW
CPT = RPT // NB
NCHUNK = TOT // NB

F_PAD = 50176
BF = 1024
BFP3 = 200
BFP4 = 1792


def _mm_body(fea_ref, w_ref, out_ref):
    x = fea_ref[0]
    w = w_ref[...]
    out_ref[0] = lax.dot_general(
        x, w, (((0,), (1,)), ((), ())), preferred_element_type=jnp.float32)


def _matmul(fea, W):
    return pl.pallas_call(
        _mm_body,
        grid=(M, F_PAD // BF),
        in_specs=[
            pl.BlockSpec((1, C_IN, BF), lambda m, j: (m, 0, j)),
            pl.BlockSpec((C_OUT, C_IN), lambda m, j: (0, 0)),
        ],
        out_specs=pl.BlockSpec((1, BF, C_OUT), lambda m, j: (m, j, 0)),
        out_shape=jax.ShapeDtypeStruct((M, F_PAD, C_OUT), jnp.float32),
    )(fea, W)


_mesh = plsc.VectorSubcoreMesh(core_axis_name="c", subcore_axis_name="s")


@functools.partial(
    pl.kernel,
    mesh=_mesh,
    out_type=jax.ShapeDtypeStruct((TOT, C_OUT), jnp.float32),
    scratch_types=[
        pltpu.VMEM((G, NB), jnp.int32),
        pltpu.VMEM((G, NB, C_OUT), jnp.float32),
        pltpu.SemaphoreType.DMA,
    ],
)
def _gather_sum(pre_hbm, idx_hbm, out_hbm, idx_v, buf_v, gsem):
    wid = lax.axis_index("s") * NC + lax.axis_index("c")

    def chunk_body(ci, carry):
        j = wid * CPT + ci
        pltpu.sync_copy(idx_hbm.at[j], idx_v)
        cps = [
            pltpu.make_async_copy(pre_hbm.at[idx_v.at[k]], buf_v.at[k], gsem)
            for k in range(G)
        ]
        for cp in cps:
            cp.start()
        for cp in cps:
            cp.wait()

        def face_body(i, c2):
            for c8 in range(8):
                sl = pl.ds(c8 * 16, 16)
                v = buf_v[0, i, sl]
                for k in range(1, G):
                    v = v + buf_v[k, i, sl]
                buf_v[0, i, sl] = v
            return c2

        lax.fori_loop(0, NB, face_body, 0, unroll=False)
        pltpu.sync_copy(buf_v.at[0], out_hbm.at[pl.ds(j * NB, NB)])
        return carry

    lax.fori_loop(0, CPT, chunk_body, 0, unroll=False)


def _stats_body(s_ref, gamma_ref, beta_ref, a_ref, c_ref, acc_ref):
    m = pl.program_id(0)
    j = pl.program_id(1)
    nj = pl.num_programs(1)

    @pl.when((m == 0) & (j == 0))
    def _():
        acc_ref[...] = jnp.zeros_like(acc_ref)

    x = s_ref[0]
    acc_ref[0:1, :] += jnp.sum(x, axis=0, keepdims=True)
    acc_ref[1:2, :] += jnp.sum(x * x, axis=0, keepdims=True)

    @pl.when((m == M - 1) & (j == nj - 1))
    def _():
        n = float(M * FP)
        mean = acc_ref[0:1, :] / n
        var = acc_ref[1:2, :] / n - mean * mean
        a = gamma_ref[...] * lax.rsqrt(var + 1e-5)
        a_ref[...] = a
        c_ref[...] = beta_ref[...] - mean * a


def _stats(s3, gamma, beta):
    return pl.pallas_call(
        _stats_body,
        grid=(M, FP // BFP3),
        in_specs=[
            pl.BlockSpec((1, BFP3, C_OUT), lambda m, j: (m, j, 0)),
            pl.BlockSpec((1, C_OUT), lambda m, j: (0, 0)),
            pl.BlockSpec((1, C_OUT), lambda m, j: (0, 0)),
        ],
        out_specs=[
            pl.BlockSpec((1, C_OUT), lambda m, j: (0, 0)),
            pl.BlockSpec((1, C_OUT), lambda m, j: (0, 0)),
        ],
        out_shape=[
            jax.ShapeDtypeStruct((1, C_OUT), jnp.float32),
            jax.ShapeDtypeStruct((1, C_OUT), jnp.float32),
        ],
        scratch_shapes=[pltpu.VMEM((8, C_OUT), jnp.float32)],
    )(s3, gamma, beta)


def _norm_body(s_ref, a_ref, c_ref, out_ref):
    x = s_ref[0]
    y = jnp.maximum(x * a_ref[...] + c_ref[...], 0.0)
    out_ref[0] = y.T


def _normalize(s3, a, c):
    return pl.pallas_call(
        _norm_body,
        grid=(M, FP_PAD // BFP4),
        in_specs=[
            pl.BlockSpec((1, BFP4, C_OUT), lambda m, j: (m, j, 0)),
            pl.BlockSpec((1, C_OUT), lambda m, j: (0, 0)),
            pl.BlockSpec((1, C_OUT), lambda m, j: (0, 0)),
        ],
        out_specs=pl.BlockSpec((1, C_OUT, BFP4), lambda m, j: (m, 0, j)),
        out_shape=jax.ShapeDtypeStruct((M, C_OUT, FP_PAD), jnp.float32),
    )(s3, a, c)


def kernel(fea, ring_n, pool_idx, W, b, gamma, beta):
    del b
    fea_p = jnp.pad(fea, ((0, 0), (0, 0), (0, F_PAD - F)))
    pre = _matmul(fea_p, W).reshape(M * F_PAD, C_OUT)

    pool_b = jnp.broadcast_to(pool_idx[None, :, None], (M, FP, 1))
    idx_all = jnp.concatenate([pool_b, ring_n], axis=2)
    idx_all = idx_all + (jnp.arange(M, dtype=jnp.int32) * F_PAD)[:, None, None]
    idx_all = jnp.pad(idx_all, ((0, 0), (0, FP_PAD - FP), (0, 0)))
    idx_chunks = idx_all.reshape(NCHUNK, NB, G).transpose(0, 2, 1)

    s = _gather_sum(pre, idx_chunks)
    s3 = s.reshape(M, FP_PAD, C_OUT)

    gamma2 = gamma.reshape(1, C_OUT)
    beta2 = beta.reshape(1, C_OUT)
    a, c = _stats(s3, gamma2, beta2)
    return _normalize(s3, a, c)[:, :, :FP]
